# baseline (speedup 1.0000x reference)
import functools

import jax
import jax.numpy as jnp
from jax import lax
from jax.experimental import pallas as pl
from jax.experimental.pallas import tpu as pltpu
from jax.experimental.pallas import tpu_sc as plsc

_VOCAB = 1000000
_DIM = 64
_B = 4096
_S = 200
_NTOK = _B * _S

_NC = 2
_NS = 16
_NW = _NC * _NS
_PER_W = _NTOK // _NW
_CHUNK = 800
_NCHUNK = _PER_W // _CHUNK

_mesh = plsc.VectorSubcoreMesh(core_axis_name="c", subcore_axis_name="s")


@functools.partial(
    pl.kernel,
    mesh=_mesh,
    out_type=jax.ShapeDtypeStruct((_NTOK, _DIM), jnp.float32),
    scratch_types=[
        pltpu.VMEM((_CHUNK,), jnp.int32),
        pltpu.VMEM((_CHUNK, _DIM), jnp.float32),
        pltpu.SemaphoreType.DMA,
    ],
    compiler_params=pltpu.CompilerParams(use_tc_tiling_on_sc=False),
)
def _emb_lookup(idx_hbm, table_hbm, out_hbm, idx_v, rows_v, sem):
    wid = lax.axis_index("s") * _NC + lax.axis_index("c")
    base = wid * _PER_W

    def body(g, carry):
        off = base + g * _CHUNK
        pltpu.sync_copy(idx_hbm.at[pl.ds(off, _CHUNK)], idx_v)
        pltpu.async_copy(table_hbm.at[idx_v], rows_v, sem).wait()
        pltpu.sync_copy(rows_v, out_hbm.at[pl.ds(off, _CHUNK)])
        return carry

    lax.fori_loop(0, _NCHUNK, body, 0)


def kernel(input_ids, tok_emb):
    flat = input_ids.reshape(_NTOK)
    out = _emb_lookup(flat, tok_emb)
    return out.reshape(_B, _S, _DIM)


# speedup vs baseline: 1.0216x; 1.0216x over previous
import functools

import jax
import jax.numpy as jnp
from jax import lax
from jax.experimental import pallas as pl
from jax.experimental.pallas import tpu as pltpu
from jax.experimental.pallas import tpu_sc as plsc

_VOCAB = 1000000
_DIM = 64
_B = 4096
_S = 200
_NTOK = _B * _S

_NC = 2
_NS = 16
_NW = _NC * _NS
_PER_W = _NTOK // _NW
_CHUNK = 800
_NCHUNK = _PER_W // _CHUNK

_mesh = plsc.VectorSubcoreMesh(core_axis_name="c", subcore_axis_name="s")


@functools.partial(
    pl.kernel,
    mesh=_mesh,
    out_type=jax.ShapeDtypeStruct((_NTOK, _DIM), jnp.float32),
    scratch_types=[
        pltpu.VMEM((_PER_W,), jnp.int32),
        pltpu.VMEM((2, _CHUNK, _DIM), jnp.float32),
        pltpu.SemaphoreType.DMA,
        pltpu.SemaphoreType.DMA,
    ],
    compiler_params=pltpu.CompilerParams(use_tc_tiling_on_sc=False),
)
def _emb_lookup(idx_hbm, table_hbm, out_hbm, idx_v, rows_v, gat_sem, st_sem):
    wid = lax.axis_index("s") * _NC + lax.axis_index("c")
    base = wid * _PER_W

    pltpu.sync_copy(idx_hbm.at[pl.ds(base, _PER_W)], idx_v)

    def gather_start(g, buf):
        pltpu.async_copy(
            table_hbm.at[idx_v.at[pl.ds(g * _CHUNK, _CHUNK)]],
            rows_v.at[buf],
            gat_sem,
        )

    def store_start(g, buf):
        pltpu.async_copy(
            rows_v.at[buf], out_hbm.at[pl.ds(base + g * _CHUNK, _CHUNK)], st_sem
        )

    gather_start(0, 0)

    def body(g, carry):
        b = g % 2
        nb = (g + 1) % 2

        @pl.when(g >= 1)
        def _():
            pltpu.make_async_copy(
                rows_v.at[nb], out_hbm.at[pl.ds(base, _CHUNK)], st_sem
            ).wait()

        @pl.when(g + 1 < _NCHUNK)
        def _():
            gather_start(g + 1, nb)

        pltpu.make_async_copy(
            table_hbm.at[idx_v.at[pl.ds(0, _CHUNK)]], rows_v.at[b], gat_sem
        ).wait()
        store_start(g, b)
        return carry

    lax.fori_loop(0, _NCHUNK, body, 0)

    pltpu.make_async_copy(
        rows_v.at[(_NCHUNK - 1) % 2], out_hbm.at[pl.ds(base, _CHUNK)], st_sem
    ).wait()


def kernel(input_ids, tok_emb):
    flat = input_ids.reshape(_NTOK)
    out = _emb_lookup(flat, tok_emb)
    return out.reshape(_B, _S, _DIM)


# speedup vs baseline: 1.0275x; 1.0058x over previous
import functools

import jax
import jax.numpy as jnp
from jax import lax
from jax.experimental import pallas as pl
from jax.experimental.pallas import tpu as pltpu
from jax.experimental.pallas import tpu_sc as plsc

_VOCAB = 1000000
_DIM = 64
_B = 4096
_S = 200

_NC = 2
_NS = 16
_NW = _NC * _NS
_ROWS_W = _B // _NW
_NBUF = 4

_mesh = plsc.VectorSubcoreMesh(core_axis_name="c", subcore_axis_name="s")


@functools.partial(
    pl.kernel,
    mesh=_mesh,
    out_type=jax.ShapeDtypeStruct((_B, _S, _DIM), jnp.float32),
    scratch_types=[
        pltpu.VMEM((_ROWS_W, _S), jnp.int32),
        pltpu.VMEM((_NBUF, _S, _DIM), jnp.float32),
        pltpu.SemaphoreType.DMA,
        pltpu.SemaphoreType.DMA,
    ],
    compiler_params=pltpu.CompilerParams(use_tc_tiling_on_sc=False),
)
def _emb_lookup(ids_hbm, table_hbm, out_hbm, idx_v, rows_v, gat_sem, st_sem):
    wid = lax.axis_index("s") * _NC + lax.axis_index("c")
    b0 = wid * _ROWS_W

    pltpu.sync_copy(ids_hbm.at[pl.ds(b0, _ROWS_W)], idx_v)

    def gather_start(g, buf):
        pltpu.async_copy(table_hbm.at[idx_v.at[g]], rows_v.at[buf], gat_sem)

    def gather_wait(g, buf):
        pltpu.make_async_copy(
            table_hbm.at[idx_v.at[g]], rows_v.at[buf], gat_sem
        ).wait()

    def store_start(g, buf):
        pltpu.async_copy(rows_v.at[buf], out_hbm.at[b0 + g], st_sem)

    def store_wait(g, buf):
        pltpu.make_async_copy(
            rows_v.at[buf], out_hbm.at[b0 + g], st_sem
        ).wait()

    for k in range(_NBUF):
        gather_start(k, k)

    def body(g, carry):
        buf = g % _NBUF
        gather_wait(g, buf)
        store_start(g, buf)

        @pl.when(g + _NBUF < _ROWS_W)
        def _():
            store_wait(g, buf)
            gather_start(g + _NBUF, buf)

        return carry

    lax.fori_loop(0, _ROWS_W, body, 0)

    for k in range(_NBUF):
        store_wait(k, k)


def kernel(input_ids, tok_emb):
    return _emb_lookup(input_ids, tok_emb)
